# baseline (device time: 173477 ns/iter reference)
import jax
import jax.numpy as jnp
from jax import lax
from jax.experimental import pallas as pl
from jax.experimental.pallas import tpu as pltpu

B, S, D = 4, 256, 4096
H, Dh, Dr = 32, 128, 64
DC_HALF = 128
N_KV = H * Dh
SCALE = (Dh + Dr) ** -0.5
_MESH = pl.DeviceIdType.MESH


def _ring_pos(my_x, my_y):
    return jnp.where(my_x == 0, my_y, 3 - my_y)


def _ring_neighbors(my_x, my_y):
    even = (my_x + my_y) % 2 == 0
    right = (jnp.where(even, my_x, 1 - my_x), jnp.where(even, 1 - my_y, my_y))
    left = (jnp.where(even, 1 - my_x, my_x), jnp.where(even, my_y, 1 - my_y))
    return left, right


def _mm_body(x_ref, w_ref, o_ref, acc_ref):
    @pl.when(pl.program_id(2) == 0)
    def _():
        acc_ref[...] = jnp.zeros_like(acc_ref)

    acc_ref[...] += jnp.dot(
        x_ref[...], w_ref[...], preferred_element_type=jnp.float32
    )

    @pl.when(pl.program_id(2) == pl.num_programs(2) - 1)
    def _():
        o_ref[...] = acc_ref[...]


def _matmul(x, w, bm=1024, bn=1024, bk=1024):
    m, k = x.shape
    _, n = w.shape
    bm, bn, bk = min(bm, m), min(bn, n), min(bk, k)
    return pl.pallas_call(
        _mm_body,
        grid=(m // bm, n // bn, k // bk),
        in_specs=[
            pl.BlockSpec((bm, bk), lambda i, j, kk: (i, kk)),
            pl.BlockSpec((bk, bn), lambda i, j, kk: (kk, j)),
        ],
        out_specs=pl.BlockSpec((bm, bn), lambda i, j, kk: (i, j)),
        out_shape=jax.ShapeDtypeStruct((m, n), jnp.float32),
        scratch_shapes=[pltpu.VMEM((bm, bn), jnp.float32)],
    )(x, w)


def _myrow_body(p_ref, x_ref, w_ref, o_ref, acc_ref):
    del p_ref
    _mm_body(x_ref, w_ref, o_ref, acc_ref)


def _matmul_myrow(x_full, w, p_arr, bm=S, bn=1024, bk=1024):
    _, k = x_full.shape
    _, n = w.shape
    bn, bk = min(bn, n), min(bk, k)
    return pl.pallas_call(
        _myrow_body,
        grid_spec=pltpu.PrefetchScalarGridSpec(
            num_scalar_prefetch=1,
            grid=(1, n // bn, k // bk),
            in_specs=[
                pl.BlockSpec((bm, bk), lambda i, j, kk, pr: (pr[0], kk)),
                pl.BlockSpec((bk, bn), lambda i, j, kk, pr: (kk, j)),
            ],
            out_specs=pl.BlockSpec((bm, bn), lambda i, j, kk, pr: (i, j)),
            scratch_shapes=[pltpu.VMEM((bm, bn), jnp.float32)],
        ),
        out_shape=jax.ShapeDtypeStruct((bm, n), jnp.float32),
    )(p_arr, x_full, w)


_QBN = 1024
_QBK = 1024
_WSUB = 1024


def _w_sub(jj, my_x):
    col = my_x * (N_KV // 2) + (jj % 2) * _WSUB
    return jj < 2, pl.ds(col, _WSUB)


def _ysend_w(refs, jj, my_y, my_x, sems_s, sems_r, nbr_y):
    wuk_ref, wuv_ref, wuko_ref, wuvo_ref = refs
    is_wuk, cols = _w_sub(jj, my_x)
    src = (wuk_ref if is_wuk else wuv_ref).at[:, cols]
    dst = (wuko_ref if is_wuk else wuvo_ref).at[
        pl.ds(my_y * DC_HALF, DC_HALF), cols]
    return pltpu.make_async_remote_copy(
        src_ref=src, dst_ref=dst,
        send_sem=sems_s.at[jj], recv_sem=sems_r.at[jj],
        device_id=nbr_y, device_id_type=_MESH,
    )


def _xfwd_w(refs, jj, my_y, my_x, sems_s, sems_r, peer_x):
    _, _, wuko_ref, wuvo_ref = refs
    is_wuk, cols = _w_sub(jj, my_x)
    region = (wuko_ref if is_wuk else wuvo_ref).at[
        pl.ds((1 - my_y) * DC_HALF, DC_HALF), cols]
    return pltpu.make_async_remote_copy(
        src_ref=region, dst_ref=region,
        send_sem=sems_s.at[jj], recv_sem=sems_r.at[jj],
        device_id=peer_x, device_id_type=_MESH,
    )


def _c_copy(cp_ref, co_ref, q_nbr, my_y, sems_s, sems_r, nbr_y):
    return pltpu.make_async_remote_copy(
        src_ref=cp_ref.at[pl.ds(q_nbr * S, S), :],
        dst_ref=co_ref.at[:, pl.ds(my_y * DC_HALF, DC_HALF)],
        send_sem=sems_s.at[4], recv_sem=sems_r.at[4],
        device_id=nbr_y, device_id_type=_MESH,
    )


def _xq_body(p_ref, x_ref, wq_ref, wqr_ref, cp_ref, wuk_ref, wuv_ref,
             q_out, co_ref, wuko_ref, wuvo_ref,
             acc_ref, ys_sems, yr_sems, xs_sems, xr_sems):
    del p_ref
    j = pl.program_id(0)
    k = pl.program_id(1)
    nj = pl.num_programs(0)
    nk = pl.num_programs(1)
    my_x = lax.axis_index("x")
    my_y = lax.axis_index("y")
    nbr_y = (my_x, 1 - my_y)
    peer_x = (1 - my_x, my_y)
    q_me = jnp.where(my_x == 0, my_y, 1 - my_y)
    q_nbr = 1 - q_me
    wrefs = (wuk_ref, wuv_ref, wuko_ref, wuvo_ref)

    @pl.when((j == 0) & (k == 0))
    def _():
        barrier = pltpu.get_barrier_semaphore()
        for nb in (nbr_y, peer_x):
            pl.semaphore_signal(
                barrier, inc=1, device_id=nb, device_id_type=_MESH
            )
        pl.semaphore_wait(barrier, 2)
        _c_copy(cp_ref, co_ref, q_nbr, my_y, ys_sems, yr_sems, nbr_y).start()
        for jj in range(4):
            _ysend_w(wrefs, jj, my_y, my_x, ys_sems, yr_sems, nbr_y).start()
        off = my_y * DC_HALF
        co_ref[:, pl.ds(off, DC_HALF)] = cp_ref[pl.ds(q_me * S, S), :]
        wuko_ref[pl.ds(off, DC_HALF), :] = wuk_ref[...]
        wuvo_ref[pl.ds(off, DC_HALF), :] = wuv_ref[...]

    @pl.when(k == 0)
    def _():
        acc_ref[...] = jnp.zeros_like(acc_ref)

    @pl.when(j < D // _QBN)
    def _():
        acc_ref[...] += jnp.dot(
            x_ref[...], wq_ref[...], preferred_element_type=jnp.float32
        )

    @pl.when(j >= D // _QBN)
    def _():
        acc_ref[...] += jnp.dot(
            x_ref[...], wqr_ref[...], preferred_element_type=jnp.float32
        )

    @pl.when(k == nk - 1)
    def _():
        q_out[...] = acc_ref[...]

    for jj in range(4):
        @pl.when((j == jj) & (k == nk - 1))
        def _(jj=jj):
            _ysend_w(wrefs, jj, my_y, my_x, ys_sems, yr_sems,
                     nbr_y).wait_recv()
            _xfwd_w(wrefs, jj, my_y, my_x, xs_sems, xr_sems, peer_x).start()

    @pl.when((j == nj - 1) & (k == nk - 1))
    def _():
        _c_copy(cp_ref, co_ref, q_nbr, my_y, ys_sems, yr_sems,
                nbr_y).wait()
        for jj in range(4):
            _xfwd_w(wrefs, jj, my_y, my_x, xs_sems, xr_sems,
                    peer_x).wait()
            _ysend_w(wrefs, jj, my_y, my_x, ys_sems, yr_sems,
                     nbr_y).wait_send()


def _q_and_y_exchange(x2, Wq, Wqr, c_all, Wuk, Wuv, p_arr):
    nq = D // _QBN
    full = lambda shape: pl.BlockSpec(
        shape, lambda j, k, pr: (0,) * len(shape)
    )
    return pl.pallas_call(
        _xq_body,
        grid_spec=pltpu.PrefetchScalarGridSpec(
            num_scalar_prefetch=1,
            grid=(nq + (H * Dr) // _QBN, D // _QBK),
            in_specs=[
                pl.BlockSpec((S, _QBK), lambda j, k, pr: (pr[0], k)),
                pl.BlockSpec(
                    (_QBK, _QBN),
                    lambda j, k, pr: (jnp.where(j < nq, k, D // _QBK - 1),
                                      jnp.where(j < nq, j, nq - 1)),
                ),
                pl.BlockSpec(
                    (_QBK, _QBN),
                    lambda j, k, pr: (jnp.where(j < nq, 0, k),
                                      jnp.where(j < nq, 0, j - nq)),
                ),
                full((2 * S, DC_HALF)),
                full((DC_HALF, N_KV)),
                full((DC_HALF, N_KV)),
            ],
            out_specs=[
                pl.BlockSpec((S, _QBN), lambda j, k, pr: (0, j)),
                full((S, 2 * DC_HALF)),
                full((2 * DC_HALF, N_KV)),
                full((2 * DC_HALF, N_KV)),
            ],
            scratch_shapes=[
                pltpu.VMEM((S, _QBN), jnp.float32),
                pltpu.SemaphoreType.DMA((5,)),
                pltpu.SemaphoreType.DMA((5,)),
                pltpu.SemaphoreType.DMA((4,)),
                pltpu.SemaphoreType.DMA((4,)),
            ],
        ),
        out_shape=(
            jax.ShapeDtypeStruct((S, D + H * Dr), jnp.float32),
            jax.ShapeDtypeStruct((S, 2 * DC_HALF), jnp.float32),
            jax.ShapeDtypeStruct((2 * DC_HALF, N_KV), jnp.float32),
            jax.ShapeDtypeStruct((2 * DC_HALF, N_KV), jnp.float32),
        ),
        compiler_params=pltpu.CompilerParams(collective_id=0),
    )(p_arr, x2, Wq, Wqr, c_all, Wuk, Wuv)


_OBN = 1024
_OBK = 1024
_HALF_S = S // 2


def _h1_copy(out_ref, jj, p_slot, sems_a, sems_b, base, tgt):
    return pltpu.make_async_remote_copy(
        src_ref=out_ref.at[pl.ds(p_slot * S, S), pl.ds(jj * _OBN, _OBN)],
        dst_ref=out_ref.at[pl.ds(p_slot * S, S), pl.ds(jj * _OBN, _OBN)],
        send_sem=sems_a.at[base + jj], recv_sem=sems_b.at[base + jj],
        device_id=tgt, device_id_type=_MESH,
    )


def _h2_copy(out_ref, jj, slot, row_off, sems_a, sems_b, base, tgt):
    return pltpu.make_async_remote_copy(
        src_ref=out_ref.at[pl.ds(slot * S + row_off, _HALF_S),
                           pl.ds(jj * _OBN, _OBN)],
        dst_ref=out_ref.at[pl.ds(slot * S + row_off, _HALF_S),
                           pl.ds(jj * _OBN, _OBN)],
        send_sem=sems_a.at[base + jj], recv_sem=sems_b.at[base + jj],
        device_id=tgt, device_id_type=_MESH,
    )


def _wo_ag_body(o2_ref, wo_ref, out_ref, acc_ref, send_sems, recv_sems):
    j = pl.program_id(0)
    k = pl.program_id(1)
    nj = pl.num_programs(0)
    nk = pl.num_programs(1)
    my_x = lax.axis_index("x")
    my_y = lax.axis_index("y")
    p = _ring_pos(my_x, my_y)
    left, right = _ring_neighbors(my_x, my_y)
    p_left = (p + 3) % 4
    p_right = (p + 1) % 4

    @pl.when((j == 0) & (k == 0))
    def _():
        barrier = pltpu.get_barrier_semaphore()
        for nb in (left, right):
            pl.semaphore_signal(
                barrier, inc=1, device_id=nb, device_id_type=_MESH
            )
        pl.semaphore_wait(barrier, 2)

    @pl.when(k == 0)
    def _():
        acc_ref[...] = jnp.zeros_like(acc_ref)

    acc_ref[...] += jnp.dot(
        o2_ref[...], wo_ref[...], preferred_element_type=jnp.float32
    )

    @pl.when(k == nk - 1)
    def _():
        out_ref[pl.ds(p * S, S), pl.ds(j * _OBN, _OBN)] = acc_ref[...]
        _h1_copy(out_ref, j, p, send_sems, recv_sems, 0, left).start()
        _h1_copy(out_ref, j, p, send_sems, recv_sems, nj, right).start()

    @pl.when((k == nk - 1) & (j >= 1))
    def _():
        jj = j - 1
        _h1_copy(out_ref, jj, p_right, send_sems, recv_sems,
                 0, left).wait_recv()
        _h2_copy(out_ref, jj, p_right, 0, send_sems, recv_sems,
                 2 * nj, left).start()
        _h1_copy(out_ref, jj, p_left, send_sems, recv_sems,
                 nj, right).wait_recv()
        _h2_copy(out_ref, jj, p_left, _HALF_S, send_sems, recv_sems,
                 3 * nj, right).start()

    @pl.when((j == nj - 1) & (k == nk - 1))
    def _():
        jj = nj - 1
        _h1_copy(out_ref, jj, p_right, send_sems, recv_sems,
                 0, left).wait_recv()
        _h2_copy(out_ref, jj, p_right, 0, send_sems, recv_sems,
                 2 * nj, left).start()
        _h1_copy(out_ref, jj, p_left, send_sems, recv_sems,
                 nj, right).wait_recv()
        _h2_copy(out_ref, jj, p_left, _HALF_S, send_sems, recv_sems,
                 3 * nj, right).start()
        for jj in range(nj):
            _h2_copy(out_ref, jj, p_right, 0, send_sems, recv_sems,
                     2 * nj, left).wait_recv()
            _h2_copy(out_ref, jj, p_left, _HALF_S, send_sems, recv_sems,
                     3 * nj, right).wait_recv()
        for jj in range(nj):
            _h1_copy(out_ref, jj, p, send_sems, recv_sems, 0, left).wait_send()
            _h1_copy(out_ref, jj, p, send_sems, recv_sems, nj, right).wait_send()
            _h2_copy(out_ref, jj, p_right, 0, send_sems, recv_sems,
                     2 * nj, left).wait_send()
            _h2_copy(out_ref, jj, p_left, _HALF_S, send_sems, recv_sems,
                     3 * nj, right).wait_send()


def _wo_and_allgather(O2, Wo):
    nj = D // _OBN
    return pl.pallas_call(
        _wo_ag_body,
        grid=(nj, (H * Dh) // _OBK),
        in_specs=[
            pl.BlockSpec((S, _OBK), lambda j, k: (0, k)),
            pl.BlockSpec((_OBK, _OBN), lambda j, k: (k, j)),
        ],
        out_specs=pl.BlockSpec((B * S, D), lambda j, k: (0, 0)),
        out_shape=jax.ShapeDtypeStruct((B * S, D), jnp.float32),
        scratch_shapes=[
            pltpu.VMEM((S, _OBN), jnp.float32),
            pltpu.SemaphoreType.DMA((4 * nj,)),
            pltpu.SemaphoreType.DMA((4 * nj,)),
        ],
        compiler_params=pltpu.CompilerParams(collective_id=1),
    )(O2, Wo)


def _attn_body(q_ref, c_ref, wuk_ref, wuv_ref, qr_ref, kr_ref, o_ref):
    c = c_ref[...]
    kr = kr_ref[...]
    qr_blk = qr_ref[...]
    for i in range(4):
        q = q_ref[:, i * Dh:(i + 1) * Dh]
        k = jnp.dot(c, wuk_ref[:, i * Dh:(i + 1) * Dh],
                    preferred_element_type=jnp.float32)
        v = jnp.dot(c, wuv_ref[:, i * Dh:(i + 1) * Dh],
                    preferred_element_type=jnp.float32)
        qr = qr_blk[:, i * Dr:(i + 1) * Dr]
        s = (
            lax.dot_general(q, k, (((1,), (1,)), ((), ())),
                            preferred_element_type=jnp.float32)
            + lax.dot_general(qr, kr, (((1,), (1,)), ((), ())),
                              preferred_element_type=jnp.float32)
        )
        pr = jnp.exp(s * SCALE)
        pr = pr * (1.0 / jnp.sum(pr, axis=-1, keepdims=True))
        o_ref[:, i * Dh:(i + 1) * Dh] = jnp.dot(
            pr, v, preferred_element_type=jnp.float32
        )


def _attention(QQr, c_me, Wuk_f, Wuv_f, Kr):
    ws = pl.BlockSpec((2 * DC_HALF, 4 * Dh), lambda g: (0, g))
    return pl.pallas_call(
        _attn_body,
        grid=(H // 4,),
        in_specs=[
            pl.BlockSpec((S, 4 * Dh), lambda g: (0, g)),
            pl.BlockSpec((S, 2 * DC_HALF), lambda g: (0, 0)),
            ws, ws,
            pl.BlockSpec((S, 4 * Dr), lambda g: (0, D // (4 * Dr) + g)),
            pl.BlockSpec((S, Dr), lambda g: (0, 0)),
        ],
        out_specs=pl.BlockSpec((S, 4 * Dh), lambda g: (0, g)),
        out_shape=jax.ShapeDtypeStruct((S, H * Dh), jnp.float32),
    )(QQr, c_me, Wuk_f, Wuv_f, QQr, Kr)


def kernel(x, Wdkv, Wuk, Wuv, Wq, Wqr, Wkr, Wo):
    my_x = lax.axis_index("x")
    my_y = lax.axis_index("y")
    p = _ring_pos(my_x, my_y)

    x2 = x.reshape(B * S, D)
    c_pair = _matmul_myrow(
        x2, Wdkv, jnp.reshape(my_x, (1,)), bm=2 * S
    )

    p_arr = jnp.reshape(p, (1,))
    QQr, c_me, Wuk_f, Wuv_f = _q_and_y_exchange(
        x2, Wq, Wqr, c_pair, Wuk, Wuv, p_arr
    )
    Kr = _matmul_myrow(x2, Wkr, p_arr)

    O2 = _attention(QQr, c_me, Wuk_f, Wuv_f, Kr)

    out = _wo_and_allgather(O2, Wo)
    return out.reshape(B, S, D)


# device time: 169149 ns/iter; 1.0256x vs baseline; 1.0256x over previous
import jax
import jax.numpy as jnp
from jax import lax
from jax.experimental import pallas as pl
from jax.experimental.pallas import tpu as pltpu

B, S, D = 4, 256, 4096
H, Dh, Dr = 32, 128, 64
DC_HALF = 128
N_KV = H * Dh
SCALE = (Dh + Dr) ** -0.5
_MESH = pl.DeviceIdType.MESH


def _ring_pos(my_x, my_y):
    return jnp.where(my_x == 0, my_y, 3 - my_y)


def _ring_neighbors(my_x, my_y):
    even = (my_x + my_y) % 2 == 0
    right = (jnp.where(even, my_x, 1 - my_x), jnp.where(even, 1 - my_y, my_y))
    left = (jnp.where(even, 1 - my_x, my_x), jnp.where(even, my_y, 1 - my_y))
    return left, right


def _mm_body(x_ref, w_ref, o_ref, acc_ref):
    @pl.when(pl.program_id(2) == 0)
    def _():
        acc_ref[...] = jnp.zeros_like(acc_ref)

    acc_ref[...] += jnp.dot(
        x_ref[...], w_ref[...], preferred_element_type=jnp.float32
    )

    @pl.when(pl.program_id(2) == pl.num_programs(2) - 1)
    def _():
        o_ref[...] = acc_ref[...]


def _matmul(x, w, bm=1024, bn=1024, bk=1024):
    m, k = x.shape
    _, n = w.shape
    bm, bn, bk = min(bm, m), min(bn, n), min(bk, k)
    return pl.pallas_call(
        _mm_body,
        grid=(m // bm, n // bn, k // bk),
        in_specs=[
            pl.BlockSpec((bm, bk), lambda i, j, kk: (i, kk)),
            pl.BlockSpec((bk, bn), lambda i, j, kk: (kk, j)),
        ],
        out_specs=pl.BlockSpec((bm, bn), lambda i, j, kk: (i, j)),
        out_shape=jax.ShapeDtypeStruct((m, n), jnp.float32),
        scratch_shapes=[pltpu.VMEM((bm, bn), jnp.float32)],
    )(x, w)


def _myrow_body(p_ref, x_ref, w_ref, o_ref, acc_ref):
    del p_ref
    _mm_body(x_ref, w_ref, o_ref, acc_ref)


def _matmul_myrow(x_full, w, p_arr, bm=S, bn=1024, bk=1024):
    _, k = x_full.shape
    _, n = w.shape
    bn, bk = min(bn, n), min(bk, k)
    return pl.pallas_call(
        _myrow_body,
        grid_spec=pltpu.PrefetchScalarGridSpec(
            num_scalar_prefetch=1,
            grid=(1, n // bn, k // bk),
            in_specs=[
                pl.BlockSpec((bm, bk), lambda i, j, kk, pr: (pr[0], kk)),
                pl.BlockSpec((bk, bn), lambda i, j, kk, pr: (kk, j)),
            ],
            out_specs=pl.BlockSpec((bm, bn), lambda i, j, kk, pr: (i, j)),
            scratch_shapes=[pltpu.VMEM((bm, bn), jnp.float32)],
        ),
        out_shape=jax.ShapeDtypeStruct((bm, n), jnp.float32),
    )(p_arr, x_full, w)


_QBN = 1024
_QBK = 1024
_WSUB = 1024


def _w_sub(jj, my_x):
    col = my_x * (N_KV // 2) + (jj % 2) * _WSUB
    return jj < 2, pl.ds(col, _WSUB)


def _ysend_w(refs, jj, my_y, my_x, sems_s, sems_r, nbr_y):
    wuk_ref, wuv_ref, wuko_ref, wuvo_ref = refs
    is_wuk, cols = _w_sub(jj, my_x)
    src = (wuk_ref if is_wuk else wuv_ref).at[:, cols]
    dst = (wuko_ref if is_wuk else wuvo_ref).at[
        pl.ds(my_y * DC_HALF, DC_HALF), cols]
    return pltpu.make_async_remote_copy(
        src_ref=src, dst_ref=dst,
        send_sem=sems_s.at[jj], recv_sem=sems_r.at[jj],
        device_id=nbr_y, device_id_type=_MESH,
    )


def _xfwd_w(refs, jj, my_y, my_x, sems_s, sems_r, peer_x):
    _, _, wuko_ref, wuvo_ref = refs
    is_wuk, cols = _w_sub(jj, my_x)
    region = (wuko_ref if is_wuk else wuvo_ref).at[
        pl.ds((1 - my_y) * DC_HALF, DC_HALF), cols]
    return pltpu.make_async_remote_copy(
        src_ref=region, dst_ref=region,
        send_sem=sems_s.at[jj], recv_sem=sems_r.at[jj],
        device_id=peer_x, device_id_type=_MESH,
    )


def _c_copy(cp_ref, co_ref, b_nbr, my_y, sems_s, sems_r, nbr_y):
    return pltpu.make_async_remote_copy(
        src_ref=cp_ref.at[pl.ds(b_nbr * S, S), :],
        dst_ref=co_ref.at[:, pl.ds(my_y * DC_HALF, DC_HALF)],
        send_sem=sems_s.at[4], recv_sem=sems_r.at[4],
        device_id=nbr_y, device_id_type=_MESH,
    )


def _xq_body(p_ref, x_ref, wq_ref, wqr_ref, cp_ref, wuk_ref, wuv_ref,
             q_out, co_ref, wuko_ref, wuvo_ref,
             acc_ref, ys_sems, yr_sems, xs_sems, xr_sems):
    del p_ref
    j = pl.program_id(0)
    k = pl.program_id(1)
    nj = pl.num_programs(0)
    nk = pl.num_programs(1)
    my_x = lax.axis_index("x")
    my_y = lax.axis_index("y")
    nbr_y = (my_x, 1 - my_y)
    peer_x = (1 - my_x, my_y)
    p = _ring_pos(my_x, my_y)
    b_nbr = 2 * my_x + jnp.where(my_x == 0, 1 - my_y, my_y)
    wrefs = (wuk_ref, wuv_ref, wuko_ref, wuvo_ref)

    @pl.when((j == 0) & (k == 0))
    def _():
        barrier = pltpu.get_barrier_semaphore()
        for nb in (nbr_y, peer_x):
            pl.semaphore_signal(
                barrier, inc=1, device_id=nb, device_id_type=_MESH
            )
        pl.semaphore_wait(barrier, 2)
        _c_copy(cp_ref, co_ref, b_nbr, my_y, ys_sems, yr_sems, nbr_y).start()
        for jj in range(4):
            _ysend_w(wrefs, jj, my_y, my_x, ys_sems, yr_sems, nbr_y).start()
        off = my_y * DC_HALF
        co_ref[:, pl.ds(off, DC_HALF)] = cp_ref[pl.ds(p * S, S), :]
        wuko_ref[pl.ds(off, DC_HALF), :] = wuk_ref[...]
        wuvo_ref[pl.ds(off, DC_HALF), :] = wuv_ref[...]

    @pl.when(k == 0)
    def _():
        acc_ref[...] = jnp.zeros_like(acc_ref)

    @pl.when(j < D // _QBN)
    def _():
        acc_ref[...] += jnp.dot(
            x_ref[...], wq_ref[...], preferred_element_type=jnp.float32
        )

    @pl.when(j >= D // _QBN)
    def _():
        acc_ref[...] += jnp.dot(
            x_ref[...], wqr_ref[...], preferred_element_type=jnp.float32
        )

    @pl.when(k == nk - 1)
    def _():
        q_out[...] = acc_ref[...]

    for jj in range(4):
        @pl.when((j == jj) & (k == nk - 1))
        def _(jj=jj):
            _ysend_w(wrefs, jj, my_y, my_x, ys_sems, yr_sems,
                     nbr_y).wait_recv()
            _xfwd_w(wrefs, jj, my_y, my_x, xs_sems, xr_sems, peer_x).start()

    @pl.when((j == nj - 1) & (k == nk - 1))
    def _():
        _c_copy(cp_ref, co_ref, b_nbr, my_y, ys_sems, yr_sems,
                nbr_y).wait()
        for jj in range(4):
            _xfwd_w(wrefs, jj, my_y, my_x, xs_sems, xr_sems,
                    peer_x).wait()
            _ysend_w(wrefs, jj, my_y, my_x, ys_sems, yr_sems,
                     nbr_y).wait_send()


def _q_and_y_exchange(x2, Wq, Wqr, c_all, Wuk, Wuv, p_arr):
    nq = D // _QBN
    full = lambda shape: pl.BlockSpec(
        shape, lambda j, k, pr: (0,) * len(shape)
    )
    return pl.pallas_call(
        _xq_body,
        grid_spec=pltpu.PrefetchScalarGridSpec(
            num_scalar_prefetch=1,
            grid=(nq + (H * Dr) // _QBN, D // _QBK),
            in_specs=[
                pl.BlockSpec((S, _QBK), lambda j, k, pr: (pr[0], k)),
                pl.BlockSpec(
                    (_QBK, _QBN),
                    lambda j, k, pr: (jnp.where(j < nq, k, D // _QBK - 1),
                                      jnp.where(j < nq, j, nq - 1)),
                ),
                pl.BlockSpec(
                    (_QBK, _QBN),
                    lambda j, k, pr: (jnp.where(j < nq, 0, k),
                                      jnp.where(j < nq, 0, j - nq)),
                ),
                full((B * S, DC_HALF)),
                full((DC_HALF, N_KV)),
                full((DC_HALF, N_KV)),
            ],
            out_specs=[
                pl.BlockSpec((S, _QBN), lambda j, k, pr: (0, j)),
                full((S, 2 * DC_HALF)),
                full((2 * DC_HALF, N_KV)),
                full((2 * DC_HALF, N_KV)),
            ],
            scratch_shapes=[
                pltpu.VMEM((S, _QBN), jnp.float32),
                pltpu.SemaphoreType.DMA((5,)),
                pltpu.SemaphoreType.DMA((5,)),
                pltpu.SemaphoreType.DMA((4,)),
                pltpu.SemaphoreType.DMA((4,)),
            ],
        ),
        out_shape=(
            jax.ShapeDtypeStruct((S, D + H * Dr), jnp.float32),
            jax.ShapeDtypeStruct((S, 2 * DC_HALF), jnp.float32),
            jax.ShapeDtypeStruct((2 * DC_HALF, N_KV), jnp.float32),
            jax.ShapeDtypeStruct((2 * DC_HALF, N_KV), jnp.float32),
        ),
        compiler_params=pltpu.CompilerParams(collective_id=0),
    )(p_arr, x2, Wq, Wqr, c_all, Wuk, Wuv)


_OBN = 1024
_OBK = 1024
_HALF_S = S // 2


def _h1_copy(out_ref, jj, p_slot, sems_a, sems_b, base, tgt):
    return pltpu.make_async_remote_copy(
        src_ref=out_ref.at[pl.ds(p_slot * S, S), pl.ds(jj * _OBN, _OBN)],
        dst_ref=out_ref.at[pl.ds(p_slot * S, S), pl.ds(jj * _OBN, _OBN)],
        send_sem=sems_a.at[base + jj], recv_sem=sems_b.at[base + jj],
        device_id=tgt, device_id_type=_MESH,
    )


def _h2_copy(out_ref, jj, slot, row_off, sems_a, sems_b, base, tgt):
    return pltpu.make_async_remote_copy(
        src_ref=out_ref.at[pl.ds(slot * S + row_off, _HALF_S),
                           pl.ds(jj * _OBN, _OBN)],
        dst_ref=out_ref.at[pl.ds(slot * S + row_off, _HALF_S),
                           pl.ds(jj * _OBN, _OBN)],
        send_sem=sems_a.at[base + jj], recv_sem=sems_b.at[base + jj],
        device_id=tgt, device_id_type=_MESH,
    )


def _wo_ag_body(o2_ref, wo_ref, out_ref, acc_ref, send_sems, recv_sems):
    j = pl.program_id(0)
    k = pl.program_id(1)
    nj = pl.num_programs(0)
    nk = pl.num_programs(1)
    my_x = lax.axis_index("x")
    my_y = lax.axis_index("y")
    p = _ring_pos(my_x, my_y)
    left, right = _ring_neighbors(my_x, my_y)
    p_left = (p + 3) % 4
    p_right = (p + 1) % 4

    @pl.when((j == 0) & (k == 0))
    def _():
        barrier = pltpu.get_barrier_semaphore()
        for nb in (left, right):
            pl.semaphore_signal(
                barrier, inc=1, device_id=nb, device_id_type=_MESH
            )
        pl.semaphore_wait(barrier, 2)

    @pl.when(k == 0)
    def _():
        acc_ref[...] = jnp.zeros_like(acc_ref)

    acc_ref[...] += jnp.dot(
        o2_ref[...], wo_ref[...], preferred_element_type=jnp.float32
    )

    @pl.when(k == nk - 1)
    def _():
        out_ref[pl.ds(p * S, S), pl.ds(j * _OBN, _OBN)] = acc_ref[...]
        _h1_copy(out_ref, j, p, send_sems, recv_sems, 0, left).start()
        _h1_copy(out_ref, j, p, send_sems, recv_sems, nj, right).start()

    @pl.when((k == nk - 1) & (j >= 1))
    def _():
        jj = j - 1
        _h1_copy(out_ref, jj, p_right, send_sems, recv_sems,
                 0, left).wait_recv()
        _h2_copy(out_ref, jj, p_right, 0, send_sems, recv_sems,
                 2 * nj, left).start()
        _h1_copy(out_ref, jj, p_left, send_sems, recv_sems,
                 nj, right).wait_recv()
        _h2_copy(out_ref, jj, p_left, _HALF_S, send_sems, recv_sems,
                 3 * nj, right).start()

    @pl.when((j == nj - 1) & (k == nk - 1))
    def _():
        jj = nj - 1
        _h1_copy(out_ref, jj, p_right, send_sems, recv_sems,
                 0, left).wait_recv()
        _h2_copy(out_ref, jj, p_right, 0, send_sems, recv_sems,
                 2 * nj, left).start()
        _h1_copy(out_ref, jj, p_left, send_sems, recv_sems,
                 nj, right).wait_recv()
        _h2_copy(out_ref, jj, p_left, _HALF_S, send_sems, recv_sems,
                 3 * nj, right).start()
        for jj in range(nj):
            _h2_copy(out_ref, jj, p_right, 0, send_sems, recv_sems,
                     2 * nj, left).wait_recv()
            _h2_copy(out_ref, jj, p_left, _HALF_S, send_sems, recv_sems,
                     3 * nj, right).wait_recv()
        for jj in range(nj):
            _h1_copy(out_ref, jj, p, send_sems, recv_sems, 0, left).wait_send()
            _h1_copy(out_ref, jj, p, send_sems, recv_sems, nj, right).wait_send()
            _h2_copy(out_ref, jj, p_right, 0, send_sems, recv_sems,
                     2 * nj, left).wait_send()
            _h2_copy(out_ref, jj, p_left, _HALF_S, send_sems, recv_sems,
                     3 * nj, right).wait_send()


def _wo_and_allgather(O2, Wo):
    nj = D // _OBN
    return pl.pallas_call(
        _wo_ag_body,
        grid=(nj, (H * Dh) // _OBK),
        in_specs=[
            pl.BlockSpec((S, _OBK), lambda j, k: (0, k)),
            pl.BlockSpec((_OBK, _OBN), lambda j, k: (k, j)),
        ],
        out_specs=pl.BlockSpec((B * S, D), lambda j, k: (0, 0)),
        out_shape=jax.ShapeDtypeStruct((B * S, D), jnp.float32),
        scratch_shapes=[
            pltpu.VMEM((S, _OBN), jnp.float32),
            pltpu.SemaphoreType.DMA((4 * nj,)),
            pltpu.SemaphoreType.DMA((4 * nj,)),
        ],
        compiler_params=pltpu.CompilerParams(collective_id=1),
    )(O2, Wo)


def _attn_body(q_ref, c_ref, wuk_ref, wuv_ref, qr_ref, kr_ref, o_ref):
    c = c_ref[...]
    kr = kr_ref[...]
    qr_blk = qr_ref[...]
    for i in range(4):
        q = q_ref[:, i * Dh:(i + 1) * Dh]
        k = jnp.dot(c, wuk_ref[:, i * Dh:(i + 1) * Dh],
                    preferred_element_type=jnp.float32)
        v = jnp.dot(c, wuv_ref[:, i * Dh:(i + 1) * Dh],
                    preferred_element_type=jnp.float32)
        qr = qr_blk[:, i * Dr:(i + 1) * Dr]
        s = (
            lax.dot_general(q, k, (((1,), (1,)), ((), ())),
                            preferred_element_type=jnp.float32)
            + lax.dot_general(qr, kr, (((1,), (1,)), ((), ())),
                              preferred_element_type=jnp.float32)
        )
        pr = jnp.exp(s * SCALE)
        pr = pr * (1.0 / jnp.sum(pr, axis=-1, keepdims=True))
        o_ref[:, i * Dh:(i + 1) * Dh] = jnp.dot(
            pr, v, preferred_element_type=jnp.float32
        )


def _attention(QQr, c_me, Wuk_f, Wuv_f, Kr):
    ws = pl.BlockSpec((2 * DC_HALF, 4 * Dh), lambda g: (0, g))
    return pl.pallas_call(
        _attn_body,
        grid=(H // 4,),
        in_specs=[
            pl.BlockSpec((S, 4 * Dh), lambda g: (0, g)),
            pl.BlockSpec((S, 2 * DC_HALF), lambda g: (0, 0)),
            ws, ws,
            pl.BlockSpec((S, 4 * Dr), lambda g: (0, D // (4 * Dr) + g)),
            pl.BlockSpec((S, Dr), lambda g: (0, 0)),
        ],
        out_specs=pl.BlockSpec((S, 4 * Dh), lambda g: (0, g)),
        out_shape=jax.ShapeDtypeStruct((S, H * Dh), jnp.float32),
    )(QQr, c_me, Wuk_f, Wuv_f, QQr, Kr)


def kernel(x, Wdkv, Wuk, Wuv, Wq, Wqr, Wkr, Wo):
    my_x = lax.axis_index("x")
    my_y = lax.axis_index("y")
    p = _ring_pos(my_x, my_y)

    x2 = x.reshape(B * S, D)
    c_all = _matmul(x2, Wdkv)

    p_arr = jnp.reshape(p, (1,))
    QQr, c_me, Wuk_f, Wuv_f = _q_and_y_exchange(
        x2, Wq, Wqr, c_all, Wuk, Wuv, p_arr
    )
    Kr = _matmul_myrow(x2, Wkr, p_arr)

    O2 = _attention(QQr, c_me, Wuk_f, Wuv_f, Kr)

    out = _wo_and_allgather(O2, Wo)
    return out.reshape(B, S, D)
